# baseline (speedup 1.0000x reference)
import functools

import jax
import jax.numpy as jnp
from jax import lax
from jax.experimental import pallas as pl
from jax.experimental.pallas import tpu as pltpu
from jax.experimental.pallas import tpu_sc as plsc

VOCAB = 100000
DIM = 1024
BATCH = 4
SEQ = 2048
ROPE_DIM = 32

_info = plsc.get_sparse_core_info()
NC, NS = _info.num_cores, _info.num_subcores
NW = NC * NS
TOTAL_ROWS = BATCH * SEQ
ROWS_PER_W = TOTAL_ROWS // NW
SEQ_PER_W = SEQ // NW
CHUNK = 32
N_CHUNKS = ROWS_PER_W // CHUNK
NBUF = 3


def _gather_body(idx_hbm, table_hbm, cos_hbm, sin_hbm, out_hbm, cos_out,
                 sin_out, idx_v, cbuf, sbuf, *rest):
    bufs = rest[:NBUF]
    gsems = rest[NBUF:2 * NBUF]
    osems = rest[2 * NBUF:3 * NBUF]
    csem, ssem = rest[3 * NBUF:3 * NBUF + 2]
    wid = lax.axis_index("s") * NC + lax.axis_index("c")
    base = wid * ROWS_PER_W
    per_batch = SEQ // ROWS_PER_W
    b = wid // per_batch
    off = (wid % per_batch) * ROWS_PER_W
    pltpu.sync_copy(idx_hbm.at[b, pl.ds(off, ROWS_PER_W)], idx_v)

    cin = pltpu.async_copy(cos_hbm.at[pl.ds(wid * SEQ_PER_W, SEQ_PER_W)],
                           cbuf, csem)
    sin_ = pltpu.async_copy(sin_hbm.at[pl.ds(wid * SEQ_PER_W, SEQ_PER_W)],
                            sbuf, ssem)

    pend_g = [None] * N_CHUNKS
    pend_o = [None] * N_CHUNKS
    for c in range(min(NBUF, N_CHUNKS)):
        pend_g[c] = pltpu.async_copy(
            table_hbm.at[idx_v.at[pl.ds(c * CHUNK, CHUNK)]], bufs[c],
            gsems[c])
    for c in range(N_CHUNKS):
        pend_g[c].wait()
        pend_o[c] = pltpu.async_copy(
            bufs[c % NBUF], out_hbm.at[pl.ds(base + c * CHUNK, CHUNK)],
            osems[c % NBUF])
        g = c + NBUF - 1
        if c >= 1 and g < N_CHUNKS:
            pend_o[c - 1].wait()
            pend_o[c - 1] = None
            pend_g[g] = pltpu.async_copy(
                table_hbm.at[idx_v.at[pl.ds(g * CHUNK, CHUNK)]],
                bufs[g % NBUF], gsems[g % NBUF])
    cin.wait()
    pltpu.sync_copy(cbuf, cos_out.at[pl.ds(wid * SEQ_PER_W, SEQ_PER_W)])
    sin_.wait()
    pltpu.sync_copy(sbuf, sin_out.at[pl.ds(wid * SEQ_PER_W, SEQ_PER_W)])
    for o in pend_o:
        if o is not None:
            o.wait()


@jax.jit
def _embedding_gather(ids, table, cos_c, sin_c):
    mesh = plsc.VectorSubcoreMesh(core_axis_name="c", subcore_axis_name="s")
    kern = functools.partial(
        pl.kernel,
        mesh=mesh,
        out_type=(
            jax.ShapeDtypeStruct((TOTAL_ROWS, DIM), jnp.float32),
            jax.ShapeDtypeStruct((SEQ, ROPE_DIM), jnp.float32),
            jax.ShapeDtypeStruct((SEQ, ROPE_DIM), jnp.float32),
        ),
        scratch_types=(
            [pltpu.VMEM((ROWS_PER_W,), jnp.int32),
             pltpu.VMEM((SEQ_PER_W, ROPE_DIM), jnp.float32),
             pltpu.VMEM((SEQ_PER_W, ROPE_DIM), jnp.float32)]
            + [pltpu.VMEM((CHUNK, DIM), jnp.float32)] * NBUF
            + [pltpu.SemaphoreType.DMA] * (2 * NBUF + 2)
        ),
    )(_gather_body)
    return kern(ids, table, cos_c, sin_c)


def kernel(input_ids, embed_tokens, cos_cached, sin_cached):
    seq_len = input_ids.shape[1]
    rows, cos, sin = _embedding_gather(input_ids, embed_tokens, cos_cached,
                                       sin_cached)
    x = rows.reshape(input_ids.shape[0], seq_len, DIM)
    return (x, cos, sin)


# speedup vs baseline: 1.1308x; 1.1308x over previous
import functools

import jax
import jax.numpy as jnp
from jax import lax
from jax.experimental import pallas as pl
from jax.experimental.pallas import tpu as pltpu
from jax.experimental.pallas import tpu_sc as plsc

VOCAB = 100000
DIM = 1024
BATCH = 4
SEQ = 2048

_info = plsc.get_sparse_core_info()
NC, NS = _info.num_cores, _info.num_subcores
NW = NC * NS
TOTAL_ROWS = BATCH * SEQ
ROWS_PER_W = TOTAL_ROWS // NW
CHUNK = 32
N_CHUNKS = ROWS_PER_W // CHUNK
NBUF = 3


def _gather_body(idx_hbm, table_hbm, out_hbm, idx_v, *rest):
    bufs = rest[:NBUF]
    gsems = rest[NBUF:2 * NBUF]
    osems = rest[2 * NBUF:3 * NBUF]
    wid = lax.axis_index("s") * NC + lax.axis_index("c")
    base = wid * ROWS_PER_W
    per_batch = SEQ // ROWS_PER_W
    b = wid // per_batch
    off = (wid % per_batch) * ROWS_PER_W
    pltpu.sync_copy(idx_hbm.at[b, pl.ds(off, ROWS_PER_W)], idx_v)

    pend_g = [None] * N_CHUNKS
    pend_o = [None] * N_CHUNKS
    for c in range(min(NBUF, N_CHUNKS)):
        pend_g[c] = pltpu.async_copy(
            table_hbm.at[idx_v.at[pl.ds(c * CHUNK, CHUNK)]], bufs[c],
            gsems[c])
    for c in range(N_CHUNKS):
        pend_g[c].wait()
        pend_o[c] = pltpu.async_copy(
            bufs[c % NBUF], out_hbm.at[pl.ds(base + c * CHUNK, CHUNK)],
            osems[c % NBUF])
        g = c + NBUF - 1
        if c >= 1 and g < N_CHUNKS:
            pend_o[c - 1].wait()
            pend_o[c - 1] = None
            pend_g[g] = pltpu.async_copy(
                table_hbm.at[idx_v.at[pl.ds(g * CHUNK, CHUNK)]],
                bufs[g % NBUF], gsems[g % NBUF])
    for o in pend_o:
        if o is not None:
            o.wait()


@jax.jit
def _embedding_gather(ids, table):
    mesh = plsc.VectorSubcoreMesh(core_axis_name="c", subcore_axis_name="s")
    kern = functools.partial(
        pl.kernel,
        mesh=mesh,
        out_type=jax.ShapeDtypeStruct((TOTAL_ROWS, DIM), jnp.float32),
        scratch_types=(
            [pltpu.VMEM((ROWS_PER_W,), jnp.int32)]
            + [pltpu.VMEM((CHUNK, DIM), jnp.float32)] * NBUF
            + [pltpu.SemaphoreType.DMA] * (2 * NBUF)
        ),
    )(_gather_body)
    return kern(ids, table)


def kernel(input_ids, embed_tokens, cos_cached, sin_cached):
    seq_len = input_ids.shape[1]
    rows = _embedding_gather(input_ids, embed_tokens)
    x = rows.reshape(input_ids.shape[0], seq_len, DIM)
    cos = cos_cached[:seq_len].astype(x.dtype)
    sin = sin_cached[:seq_len].astype(x.dtype)
    return (x, cos, sin)


# speedup vs baseline: 1.1743x; 1.0385x over previous
import functools

import jax
import jax.numpy as jnp
from jax import lax
from jax.experimental import pallas as pl
from jax.experimental.pallas import tpu as pltpu
from jax.experimental.pallas import tpu_sc as plsc

VOCAB = 100000
DIM = 1024
BATCH = 4
SEQ = 2048

_info = plsc.get_sparse_core_info()
NC, NS = _info.num_cores, _info.num_subcores
NW = NC * NS
TOTAL_ROWS = BATCH * SEQ
ROWS_PER_W = TOTAL_ROWS // NW
CHUNK = 16
N_CHUNKS = ROWS_PER_W // CHUNK
NBUF = 6


def _gather_body(idx_hbm, table_hbm, out_hbm, idx_v, *rest):
    bufs = rest[:NBUF]
    gsems = rest[NBUF:2 * NBUF]
    osems = rest[2 * NBUF:3 * NBUF]
    wid = lax.axis_index("s") * NC + lax.axis_index("c")
    base = wid * ROWS_PER_W
    per_batch = SEQ // ROWS_PER_W
    b = wid // per_batch
    off = (wid % per_batch) * ROWS_PER_W
    pltpu.sync_copy(idx_hbm.at[b, pl.ds(off, ROWS_PER_W)], idx_v)

    pend_g = [None] * N_CHUNKS
    pend_o = [None] * N_CHUNKS
    for c in range(min(NBUF, N_CHUNKS)):
        pend_g[c] = pltpu.async_copy(
            table_hbm.at[idx_v.at[pl.ds(c * CHUNK, CHUNK)]], bufs[c],
            gsems[c])
    for c in range(N_CHUNKS):
        pend_g[c].wait()
        pend_o[c] = pltpu.async_copy(
            bufs[c % NBUF], out_hbm.at[pl.ds(base + c * CHUNK, CHUNK)],
            osems[c % NBUF])
        g = c + NBUF - 1
        if c >= 1 and g < N_CHUNKS:
            pend_o[c - 1].wait()
            pend_o[c - 1] = None
            pend_g[g] = pltpu.async_copy(
                table_hbm.at[idx_v.at[pl.ds(g * CHUNK, CHUNK)]],
                bufs[g % NBUF], gsems[g % NBUF])
    for o in pend_o:
        if o is not None:
            o.wait()


@jax.jit
def _embedding_gather(ids, table):
    mesh = plsc.VectorSubcoreMesh(core_axis_name="c", subcore_axis_name="s")
    kern = functools.partial(
        pl.kernel,
        mesh=mesh,
        out_type=jax.ShapeDtypeStruct((TOTAL_ROWS, DIM), jnp.float32),
        scratch_types=(
            [pltpu.VMEM((ROWS_PER_W,), jnp.int32)]
            + [pltpu.VMEM((CHUNK, DIM), jnp.float32)] * NBUF
            + [pltpu.SemaphoreType.DMA] * (2 * NBUF)
        ),
    )(_gather_body)
    return kern(ids, table)


def kernel(input_ids, embed_tokens, cos_cached, sin_cached):
    seq_len = input_ids.shape[1]
    rows = _embedding_gather(input_ids, embed_tokens)
    x = rows.reshape(input_ids.shape[0], seq_len, DIM)
    cos = cos_cached[:seq_len].astype(x.dtype)
    sin = sin_cached[:seq_len].astype(x.dtype)
    return (x, cos, sin)
